# baseline (speedup 1.0000x reference)
import functools

import jax
import jax.numpy as jnp
from jax import lax
from jax.experimental import pallas as pl
from jax.experimental.pallas import tpu as pltpu
from jax.experimental.pallas import tpu_sc as plsc

N_NODES = 10000
N_EDGES = 160000
NFEAT = 256
NHID = 256

NC = 2
NS = 16
NW = NC * NS
ROWS_PT = 320
ACC_ROWS = 328
TRASH = 320
XROWS = 10240
SRE = 2048
NSR = -(-N_EDGES // SRE)
EPAD = NSR * SRE
W16 = SRE // 16


def _prefix16(mi, row_iota):
    x = mi
    for k in (1, 2, 4, 8):
        idx = jnp.maximum(row_iota - k, 0)
        shifted = x.at[idx].get(mode="promise_in_bounds")
        x = jnp.where(row_iota >= k, x + shifted, x)
    return x


def _splat_last(x, row_iota):
    return x.at[row_iota * 0 + 15].get(mode="promise_in_bounds")


def _sc_segment_sum(src, dst, x, zeros):
    mesh = plsc.VectorSubcoreMesh(
        core_axis_name="c", subcore_axis_name="s", num_cores=NC, num_subcores=NS
    )

    @functools.partial(
        pl.kernel,
        mesh=mesh,
        compiler_params=pltpu.CompilerParams(needs_layout_passes=False),
        out_type=jax.ShapeDtypeStruct((NW * ROWS_PT, NFEAT), jnp.float32),
        scratch_types=[
            pltpu.VMEM((SRE,), jnp.int32),
            pltpu.VMEM((SRE,), jnp.int32),
            pltpu.VMEM((SRE + 80,), jnp.int32),
            pltpu.VMEM((SRE + 80,), jnp.int32),
            pltpu.VMEM((16, NFEAT // 2), jnp.int32),
            pltpu.VMEM((16, NFEAT // 2), jnp.int32),
            pltpu.VMEM((16, NFEAT // 2), jnp.int32),
            pltpu.VMEM((16, NFEAT // 2), jnp.int32),
            pltpu.VMEM((ACC_ROWS * NFEAT,), jnp.float32),
            pltpu.VMEM((40, NFEAT), jnp.float32),
            pltpu.SemaphoreType.DMA,
            pltpu.SemaphoreType.DMA,
            pltpu.SemaphoreType.DMA,
            pltpu.SemaphoreType.DMA,
            pltpu.SemaphoreType.DMA,
            pltpu.SemaphoreType.DMA,
        ],
    )
    def k(src_hbm, dst_hbm, x_hbm, zeros_hbm, out_hbm,
          esrc, edst, psrc, pdloc, gbufa, gbufb, gbufc, gbufd, acc, obuf,
          sema, semb, semc, semd, semi, semj):
        c = lax.axis_index("c")
        s = lax.axis_index("s")
        w = s * NC + c
        lo = w * ROWS_PT
        row_iota = lax.iota(jnp.int32, 16)
        zero16 = row_iota * 0
        junk = zero16 + (SRE + 72)
        trash = zero16 + TRASH

        pltpu.sync_copy(zeros_hbm, acc)

        pltpu.sync_copy(src_hbm.at[pl.ds(0, SRE)], esrc)
        pltpu.sync_copy(dst_hbm.at[pl.ds(0, SRE)], edst)

        def sr_body(sr, carry0):
            nchunks = jnp.minimum(W16, (N_EDGES - sr * SRE + 15) // 16)

            def chunk4(i4, cnt_v):
                parts = []
                for u in range(4):
                    o = pl.multiple_of(i4 * 64 + u * 16, 16)
                    s16 = esrc[pl.ds(o, 16)]
                    d16 = edst[pl.ds(o, 16)]
                    m = (d16 >= lo) & (d16 < lo + ROWS_PT)
                    pref = _prefix16(m.astype(jnp.int32), row_iota)
                    parts.append((s16, d16, m, pref))
                base = cnt_v
                for s16, d16, m, pref in parts:
                    pos = jnp.where(m, base + pref - 1, junk)
                    plsc.store_scatter(psrc, [pos], s16)
                    plsc.store_scatter(pdloc, [pos], d16 - lo)
                    base = base + _splat_last(pref, row_iota)
                return base

            with jax.named_scope("sc_scan"):
                cnt_v = lax.fori_loop(0, nchunks // 4, chunk4, zero16)
            plsc.store_scatter(psrc, [cnt_v + row_iota], zero16)
            plsc.store_scatter(pdloc, [cnt_v + row_iota], trash)
            t16 = (cnt_v[0] + 15) // 16

            e1 = pl.multiple_of((sr + 1) * SRE, SRE)

            @pl.when(sr + 1 < NSR)
            def _prefetch():
                pltpu.async_copy(src_hbm.at[pl.ds(e1, SRE)], esrc, semi)
                pltpu.async_copy(dst_hbm.at[pl.ds(e1, SRE)], edst, semj)

            def _start(q, gb, sem_):
                qo = pl.multiple_of(q * 16, 16)
                idxv = psrc[pl.ds(qo, 16)]
                pltpu.async_copy(x_hbm.at[idxv], gb, sem_)

            def _finish(q, gb, sem_):
                qo = pl.multiple_of(q * 16, 16)
                idxv = psrc[pl.ds(qo, 16)]
                pltpu.make_async_copy(x_hbm.at[idxv], gb, sem_).wait()
                dl16 = pdloc[pl.ds(qo, 16)]
                for j in range(16):
                    base = pl.multiple_of(dl16[j] * NFEAT, NFEAT)
                    words = [gb[j, pl.ds(kk * 16, 16)]
                             for kk in range(NFEAT // 32)]
                    for kk in range(NFEAT // 32):
                        v32 = plsc.bitcast(words[kk], jnp.bfloat16)
                        a, b = plsc.unpack(
                            v32, format=plsc.PackFormat.INTERLEAVED,
                            preferred_element_type=jnp.float32)
                        plsc.addupdate(acc.at[pl.ds(base + kk * 32, 16)], a)
                        plsc.addupdate(
                            acc.at[pl.ds(base + kk * 32 + 16, 16)], b)

            ring = ((gbufa, sema), (gbufb, semb), (gbufc, semc), (gbufd, semd))

            for b in range(3):
                @pl.when(t16 > b)
                def _prime(b=b):
                    _start(b, *ring[b])

            def flush(q, carry1):
                p = lax.rem(q, 4)
                for b in range(4):
                    @pl.when((q + 3 < t16) & (lax.rem(q + 3, 4) == b))
                    def _(b=b):
                        _start(q + 3, *ring[b])
                for b in range(4):
                    @pl.when(p == b)
                    def _(b=b):
                        _finish(q, *ring[b])
                return carry1

            with jax.named_scope("sc_flush"):
                lax.fori_loop(0, t16, flush, 0)

            @pl.when(sr + 1 < NSR)
            def _wait_prefetch():
                pltpu.make_async_copy(src_hbm.at[pl.ds(e1, SRE)], esrc,
                                      semi).wait()
                pltpu.make_async_copy(dst_hbm.at[pl.ds(e1, SRE)], edst,
                                      semj).wait()

            return carry0

        lax.fori_loop(0, NSR, sr_body, 0)
        def out_chunk(c8, carry):
            cbase = pl.multiple_of(c8 * (40 * NFEAT), 40 * NFEAT)
            for r in range(40):
                for kk in range(NFEAT // 16):
                    obuf[r, pl.ds(kk * 16, 16)] = (
                        acc[pl.ds(cbase + r * NFEAT + kk * 16, 16)])
            ob = pl.multiple_of(w * ROWS_PT + c8 * 40, 8)
            pltpu.sync_copy(obuf, out_hbm.at[pl.ds(ob, 40)])
            return carry

        lax.fori_loop(0, ROWS_PT // 40, out_chunk, 0)

    return k(src, dst, x, zeros)


def _spectral_normalize(W):
    n, m = W.shape
    u = jnp.full((1, n), 1.0 / (float(n) ** 0.5), jnp.float32)
    v = jnp.full((1, m), 1.0 / (float(m) ** 0.5), jnp.float32)

    def it(i, uv):
        u, v = uv
        v = jnp.dot(u, W, preferred_element_type=jnp.float32)
        v = v / (jnp.sqrt(jnp.sum(v * v)) + 1e-12)
        u = lax.dot_general(v, W, (((1,), (1,)), ((), ())),
                            preferred_element_type=jnp.float32)
        u = u / (jnp.sqrt(jnp.sum(u * u)) + 1e-12)
        return (u, v)

    u, v = lax.fori_loop(0, 30, it, (u, v))
    sigma = jnp.sum(jnp.dot(u, W, preferred_element_type=jnp.float32) * v)
    return W / sigma


def _tc_body(x_ref, ap_ref, W1_ref, b1_ref, g_ref, be_ref, W2_ref, b2_ref, o_ref):
    W1n = _spectral_normalize(W1_ref[...])
    W2n = _spectral_normalize(W2_ref[...])
    h = x_ref[...] + ap_ref[0:N_NODES]
    h = lax.dot_general(h, W1n, (((1,), (1,)), ((), ())),
                        preferred_element_type=jnp.float32) + b1_ref[...]
    h = jnp.maximum(h, 0.0)
    mean = jnp.mean(h, axis=0, keepdims=True)
    var = jnp.mean(h * h, axis=0, keepdims=True) - mean * mean
    h = (h - mean) / jnp.sqrt(var + 1e-5) * g_ref[...] + be_ref[...]
    o_ref[...] = lax.dot_general(h, W2n, (((1,), (1,)), ((), ())),
                                 preferred_element_type=jnp.float32) + b2_ref[...]


def _tc_mlp(x, aggr_padded, W1, b1, gamma, beta, W2, b2):
    return pl.pallas_call(
        _tc_body,
        out_shape=jax.ShapeDtypeStruct((N_NODES, NHID), jnp.float32),
    )(x, aggr_padded, W1, b1.reshape(1, -1), gamma.reshape(1, -1),
      beta.reshape(1, -1), W2, b2.reshape(1, -1))


def kernel(x, edge_index, W1, b1, gamma, beta, W2, b2):
    src = jnp.pad(edge_index[0], (0, EPAD - N_EDGES))
    dst = jnp.pad(edge_index[1], (0, EPAD - N_EDGES))
    zeros = jnp.zeros((ACC_ROWS * NFEAT,), jnp.float32)
    xb = x.astype(jnp.bfloat16)
    xb = xb.reshape(N_NODES, NFEAT // 32, 2, 16).transpose(0, 1, 3, 2)
    xpk = lax.bitcast_convert_type(
        xb.reshape(N_NODES, NFEAT // 2, 2), jnp.int32)
    xpk = jnp.pad(xpk, ((0, XROWS - N_NODES), (0, 0)))
    aggr = _sc_segment_sum(src, dst, xpk, zeros)
    return _tc_mlp(x, aggr, W1, b1, gamma, beta, W2, b2)


# speedup vs baseline: 1.6377x; 1.6377x over previous
import functools

import jax
import jax.numpy as jnp
from jax import lax
from jax.experimental import pallas as pl
from jax.experimental.pallas import tpu as pltpu
from jax.experimental.pallas import tpu_sc as plsc

N_NODES = 10000
N_EDGES = 160000
NFEAT = 256
NHID = 256

NC = 2
NS = 16
NW = NC * NS
ROWS_PT = 320
ACC_ROWS = 328
TRASH = 320
XROWS = 10240
SRE = 4096
NSR = -(-N_EDGES // SRE)
EPAD = NSR * SRE
W16 = SRE // 16


def _prefix16(mi, row_iota):
    x = mi
    for k in (1, 2, 4, 8):
        idx = jnp.maximum(row_iota - k, 0)
        shifted = x.at[idx].get(mode="promise_in_bounds")
        x = jnp.where(row_iota >= k, x + shifted, x)
    return x


def _splat_last(x, row_iota):
    return x.at[row_iota * 0 + 15].get(mode="promise_in_bounds")


def _sc_segment_sum(src, dst, x, zeros):
    mesh = plsc.VectorSubcoreMesh(
        core_axis_name="c", subcore_axis_name="s", num_cores=NC, num_subcores=NS
    )

    @functools.partial(
        pl.kernel,
        mesh=mesh,
        compiler_params=pltpu.CompilerParams(needs_layout_passes=False),
        out_type=jax.ShapeDtypeStruct((NW * ROWS_PT, NFEAT), jnp.float32),
        scratch_types=[
            pltpu.VMEM((SRE,), jnp.int32),
            pltpu.VMEM((SRE,), jnp.int32),
            pltpu.VMEM((SRE + 80,), jnp.int32),
            pltpu.VMEM((SRE + 80,), jnp.int32),
            pltpu.VMEM((16, NFEAT // 2), jnp.int32),
            pltpu.VMEM((16, NFEAT // 2), jnp.int32),
            pltpu.VMEM((16, NFEAT // 2), jnp.int32),
            pltpu.VMEM((16, NFEAT // 2), jnp.int32),
            pltpu.VMEM((ACC_ROWS * NFEAT,), jnp.float32),
            pltpu.VMEM((40, NFEAT), jnp.float32),
            pltpu.SemaphoreType.DMA,
            pltpu.SemaphoreType.DMA,
            pltpu.SemaphoreType.DMA,
            pltpu.SemaphoreType.DMA,
            pltpu.SemaphoreType.DMA,
            pltpu.SemaphoreType.DMA,
        ],
    )
    def k(src_hbm, dst_hbm, x_hbm, zeros_hbm, out_hbm,
          esrc, edst, psrc, pdloc, gbufa, gbufb, gbufc, gbufd, acc, obuf,
          sema, semb, semc, semd, semi, semj):
        c = lax.axis_index("c")
        s = lax.axis_index("s")
        w = s * NC + c
        lo = w * ROWS_PT
        row_iota = lax.iota(jnp.int32, 16)
        zero16 = row_iota * 0
        junk = zero16 + (SRE + 72)
        trash = zero16 + TRASH

        pltpu.sync_copy(zeros_hbm, acc)

        pltpu.sync_copy(src_hbm.at[pl.ds(0, SRE)], esrc)
        pltpu.sync_copy(dst_hbm.at[pl.ds(0, SRE)], edst)

        def sr_body(sr, carry0):
            nchunks = jnp.minimum(W16, (N_EDGES - sr * SRE + 15) // 16)

            def chunk4(i4, cnt_v):
                parts = []
                for u in range(4):
                    o = pl.multiple_of(i4 * 64 + u * 16, 16)
                    s16 = esrc[pl.ds(o, 16)]
                    d16 = edst[pl.ds(o, 16)]
                    m = (d16 >= lo) & (d16 < lo + ROWS_PT)
                    pref = _prefix16(m.astype(jnp.int32), row_iota)
                    parts.append((s16, d16, m, pref))
                base = cnt_v
                for s16, d16, m, pref in parts:
                    pos = jnp.where(m, base + pref - 1, junk)
                    plsc.store_scatter(psrc, [pos], s16)
                    plsc.store_scatter(pdloc, [pos], d16 - lo)
                    base = base + _splat_last(pref, row_iota)
                return base

            with jax.named_scope("sc_scan"):
                cnt_v = lax.fori_loop(0, nchunks // 4, chunk4, zero16)
            plsc.store_scatter(psrc, [cnt_v + row_iota], zero16)
            plsc.store_scatter(pdloc, [cnt_v + row_iota], trash)
            t16 = (cnt_v[0] + 15) // 16

            e1 = pl.multiple_of((sr + 1) * SRE, SRE)

            @pl.when(sr + 1 < NSR)
            def _prefetch():
                pltpu.async_copy(src_hbm.at[pl.ds(e1, SRE)], esrc, semi)
                pltpu.async_copy(dst_hbm.at[pl.ds(e1, SRE)], edst, semj)

            def _start(q, gb, sem_):
                qo = pl.multiple_of(q * 16, 16)
                idxv = psrc[pl.ds(qo, 16)]
                pltpu.async_copy(x_hbm.at[idxv], gb, sem_)

            def _finish(q, gb, sem_):
                qo = pl.multiple_of(q * 16, 16)
                idxv = psrc[pl.ds(qo, 16)]
                pltpu.make_async_copy(x_hbm.at[idxv], gb, sem_).wait()
                dl16 = pdloc[pl.ds(qo, 16)]
                for j in range(16):
                    base = pl.multiple_of(dl16[j] * NFEAT, NFEAT)
                    words = [gb[j, pl.ds(kk * 16, 16)]
                             for kk in range(NFEAT // 32)]
                    for kk in range(NFEAT // 32):
                        v32 = plsc.bitcast(words[kk], jnp.bfloat16)
                        a, b = plsc.unpack(
                            v32, format=plsc.PackFormat.INTERLEAVED,
                            preferred_element_type=jnp.float32)
                        plsc.addupdate(acc.at[pl.ds(base + kk * 32, 16)], a)
                        plsc.addupdate(
                            acc.at[pl.ds(base + kk * 32 + 16, 16)], b)

            ring = ((gbufa, sema), (gbufb, semb), (gbufc, semc), (gbufd, semd))

            for b in range(3):
                @pl.when(t16 > b)
                def _prime(b=b):
                    _start(b, *ring[b])

            def flush(q, carry1):
                p = lax.rem(q, 4)
                for b in range(4):
                    @pl.when((q + 3 < t16) & (lax.rem(q + 3, 4) == b))
                    def _(b=b):
                        _start(q + 3, *ring[b])
                for b in range(4):
                    @pl.when(p == b)
                    def _(b=b):
                        _finish(q, *ring[b])
                return carry1

            with jax.named_scope("sc_flush"):
                lax.fori_loop(0, t16, flush, 0)

            @pl.when(sr + 1 < NSR)
            def _wait_prefetch():
                pltpu.make_async_copy(src_hbm.at[pl.ds(e1, SRE)], esrc,
                                      semi).wait()
                pltpu.make_async_copy(dst_hbm.at[pl.ds(e1, SRE)], edst,
                                      semj).wait()

            return carry0

        lax.fori_loop(0, NSR, sr_body, 0)
        def out_chunk(c8, carry):
            cbase = pl.multiple_of(c8 * (40 * NFEAT), 40 * NFEAT)
            for r in range(40):
                for kk in range(NFEAT // 16):
                    obuf[r, pl.ds(kk * 16, 16)] = (
                        acc[pl.ds(cbase + r * NFEAT + kk * 16, 16)])
            ob = pl.multiple_of(w * ROWS_PT + c8 * 40, 8)
            pltpu.sync_copy(obuf, out_hbm.at[pl.ds(ob, 40)])
            return carry

        lax.fori_loop(0, ROWS_PT // 40, out_chunk, 0)

    return k(src, dst, x, zeros)


def _spectral_normalize(W):
    n, m = W.shape
    u = jnp.full((1, n), 1.0 / (float(n) ** 0.5), jnp.float32)
    v = jnp.full((1, m), 1.0 / (float(m) ** 0.5), jnp.float32)

    def it(i, uv):
        u, v = uv
        v = jnp.dot(u, W, preferred_element_type=jnp.float32)
        v = v / (jnp.sqrt(jnp.sum(v * v)) + 1e-12)
        u = lax.dot_general(v, W, (((1,), (1,)), ((), ())),
                            preferred_element_type=jnp.float32)
        u = u / (jnp.sqrt(jnp.sum(u * u)) + 1e-12)
        return (u, v)

    u, v = lax.fori_loop(0, 30, it, (u, v))
    sigma = jnp.sum(jnp.dot(u, W, preferred_element_type=jnp.float32) * v)
    return W / sigma


def _tc_body(x_ref, ap_ref, W1_ref, b1_ref, g_ref, be_ref, W2_ref, b2_ref, o_ref):
    W1n = _spectral_normalize(W1_ref[...])
    W2n = _spectral_normalize(W2_ref[...])
    h = x_ref[...] + ap_ref[0:N_NODES]
    h = lax.dot_general(h, W1n, (((1,), (1,)), ((), ())),
                        preferred_element_type=jnp.float32) + b1_ref[...]
    h = jnp.maximum(h, 0.0)
    mean = jnp.mean(h, axis=0, keepdims=True)
    var = jnp.mean(h * h, axis=0, keepdims=True) - mean * mean
    h = (h - mean) / jnp.sqrt(var + 1e-5) * g_ref[...] + be_ref[...]
    o_ref[...] = lax.dot_general(h, W2n, (((1,), (1,)), ((), ())),
                                 preferred_element_type=jnp.float32) + b2_ref[...]


def _tc_mlp(x, aggr_padded, W1, b1, gamma, beta, W2, b2):
    return pl.pallas_call(
        _tc_body,
        out_shape=jax.ShapeDtypeStruct((N_NODES, NHID), jnp.float32),
    )(x, aggr_padded, W1, b1.reshape(1, -1), gamma.reshape(1, -1),
      beta.reshape(1, -1), W2, b2.reshape(1, -1))


def kernel(x, edge_index, W1, b1, gamma, beta, W2, b2):
    src = jnp.pad(edge_index[0], (0, EPAD - N_EDGES))
    dst = jnp.pad(edge_index[1], (0, EPAD - N_EDGES))
    zeros = jnp.zeros((ACC_ROWS * NFEAT,), jnp.float32)
    xb = x.astype(jnp.bfloat16)
    xb = xb.reshape(N_NODES, NFEAT // 32, 2, 16).transpose(0, 1, 3, 2)
    xpk = lax.bitcast_convert_type(
        xb.reshape(N_NODES, NFEAT // 2, 2), jnp.int32)
    xpk = jnp.pad(xpk, ((0, XROWS - N_NODES), (0, 0)))
    aggr = _sc_segment_sum(src, dst, xpk, zeros)
    return _tc_mlp(x, aggr, W1, b1, gamma, beta, W2, b2)


# speedup vs baseline: 2.3791x; 1.4527x over previous
import functools

import jax
import jax.numpy as jnp
from jax import lax
from jax.experimental import pallas as pl
from jax.experimental.pallas import tpu as pltpu
from jax.experimental.pallas import tpu_sc as plsc

N_NODES = 10000
N_EDGES = 160000
NFEAT = 256
NHID = 256

NC = 2
NS = 16
NW = NC * NS
ROWS_PT = 320
ACC_ROWS = 328
TRASH = 320
XROWS = 10240
SRE = 8192
NSR = -(-N_EDGES // SRE)
EPAD = NSR * SRE
W16 = SRE // 16


def _prefix16(mi, row_iota):
    x = mi
    for k in (1, 2, 4, 8):
        idx = jnp.maximum(row_iota - k, 0)
        shifted = x.at[idx].get(mode="promise_in_bounds")
        x = jnp.where(row_iota >= k, x + shifted, x)
    return x


def _splat_last(x, row_iota):
    return x.at[row_iota * 0 + 15].get(mode="promise_in_bounds")


def _sc_segment_sum(src, dst, x, zeros):
    mesh = plsc.VectorSubcoreMesh(
        core_axis_name="c", subcore_axis_name="s", num_cores=NC, num_subcores=NS
    )

    @functools.partial(
        pl.kernel,
        mesh=mesh,
        compiler_params=pltpu.CompilerParams(needs_layout_passes=False),
        out_type=jax.ShapeDtypeStruct((NW * ROWS_PT, NFEAT), jnp.float32),
        scratch_types=[
            pltpu.VMEM((SRE,), jnp.int32),
            pltpu.VMEM((SRE,), jnp.int32),
            pltpu.VMEM((SRE + 80,), jnp.int32),
            pltpu.VMEM((SRE + 80,), jnp.int32),
            pltpu.VMEM((16, NFEAT // 2), jnp.int32),
            pltpu.VMEM((16, NFEAT // 2), jnp.int32),
            pltpu.VMEM((16, NFEAT // 2), jnp.int32),
            pltpu.VMEM((16, NFEAT // 2), jnp.int32),
            pltpu.VMEM((ACC_ROWS * NFEAT,), jnp.float32),
            pltpu.VMEM((8, NFEAT), jnp.float32),
            pltpu.SemaphoreType.DMA,
            pltpu.SemaphoreType.DMA,
            pltpu.SemaphoreType.DMA,
            pltpu.SemaphoreType.DMA,
            pltpu.SemaphoreType.DMA,
            pltpu.SemaphoreType.DMA,
        ],
    )
    def k(src_hbm, dst_hbm, x_hbm, zeros_hbm, out_hbm,
          esrc, edst, psrc, pdloc, gbufa, gbufb, gbufc, gbufd, acc, obuf,
          sema, semb, semc, semd, semi, semj):
        c = lax.axis_index("c")
        s = lax.axis_index("s")
        w = s * NC + c
        lo = w * ROWS_PT
        row_iota = lax.iota(jnp.int32, 16)
        zero16 = row_iota * 0
        junk = zero16 + (SRE + 72)
        trash = zero16 + TRASH

        pltpu.sync_copy(zeros_hbm, acc)

        pltpu.sync_copy(src_hbm.at[pl.ds(0, SRE)], esrc)
        pltpu.sync_copy(dst_hbm.at[pl.ds(0, SRE)], edst)

        def sr_body(sr, carry0):
            nchunks = jnp.minimum(W16, (N_EDGES - sr * SRE + 15) // 16)

            def chunk4(i4, cnt_v):
                parts = []
                for u in range(4):
                    o = pl.multiple_of(i4 * 64 + u * 16, 16)
                    s16 = esrc[pl.ds(o, 16)]
                    d16 = edst[pl.ds(o, 16)]
                    m = (d16 >= lo) & (d16 < lo + ROWS_PT)
                    pref = _prefix16(m.astype(jnp.int32), row_iota)
                    parts.append((s16, d16, m, pref))
                base = cnt_v
                for s16, d16, m, pref in parts:
                    pos = jnp.where(m, base + pref - 1, junk)
                    plsc.store_scatter(psrc, [pos], s16)
                    plsc.store_scatter(pdloc, [pos], d16 - lo)
                    base = base + _splat_last(pref, row_iota)
                return base

            with jax.named_scope("sc_scan"):
                cnt_v = lax.fori_loop(0, nchunks // 4, chunk4, zero16)
            plsc.store_scatter(psrc, [cnt_v + row_iota], zero16)
            plsc.store_scatter(pdloc, [cnt_v + row_iota], trash)
            t16 = (cnt_v[0] + 15) // 16

            e1 = pl.multiple_of((sr + 1) * SRE, SRE)

            @pl.when(sr + 1 < NSR)
            def _prefetch():
                pltpu.async_copy(src_hbm.at[pl.ds(e1, SRE)], esrc, semi)
                pltpu.async_copy(dst_hbm.at[pl.ds(e1, SRE)], edst, semj)

            def _start(q, gb, sem_):
                qo = pl.multiple_of(q * 16, 16)
                idxv = psrc[pl.ds(qo, 16)]
                pltpu.async_copy(x_hbm.at[idxv], gb, sem_)

            def _finish(q, gb, sem_):
                qo = pl.multiple_of(q * 16, 16)
                idxv = psrc[pl.ds(qo, 16)]
                pltpu.make_async_copy(x_hbm.at[idxv], gb, sem_).wait()
                dl16 = pdloc[pl.ds(qo, 16)]
                for j in range(16):
                    base = pl.multiple_of(dl16[j] * NFEAT, NFEAT)
                    words = [gb[j, pl.ds(kk * 16, 16)]
                             for kk in range(NFEAT // 32)]
                    for kk in range(NFEAT // 32):
                        v32 = plsc.bitcast(words[kk], jnp.bfloat16)
                        a, b = plsc.unpack(
                            v32, format=plsc.PackFormat.INTERLEAVED,
                            preferred_element_type=jnp.float32)
                        plsc.addupdate(acc.at[pl.ds(base + kk * 32, 16)], a)
                        plsc.addupdate(
                            acc.at[pl.ds(base + kk * 32 + 16, 16)], b)

            ring = ((gbufa, sema), (gbufb, semb), (gbufc, semc), (gbufd, semd))

            for b in range(3):
                @pl.when(t16 > b)
                def _prime(b=b):
                    _start(b, *ring[b])

            def flush(q, carry1):
                p = lax.rem(q, 4)
                for b in range(4):
                    @pl.when((q + 3 < t16) & (lax.rem(q + 3, 4) == b))
                    def _(b=b):
                        _start(q + 3, *ring[b])
                for b in range(4):
                    @pl.when(p == b)
                    def _(b=b):
                        _finish(q, *ring[b])
                return carry1

            with jax.named_scope("sc_flush"):
                lax.fori_loop(0, t16, flush, 0)

            @pl.when(sr + 1 < NSR)
            def _wait_prefetch():
                pltpu.make_async_copy(src_hbm.at[pl.ds(e1, SRE)], esrc,
                                      semi).wait()
                pltpu.make_async_copy(dst_hbm.at[pl.ds(e1, SRE)], edst,
                                      semj).wait()

            return carry0

        lax.fori_loop(0, NSR, sr_body, 0)
        def out_chunk(c8, carry):
            cbase = pl.multiple_of(c8 * (8 * NFEAT), 8 * NFEAT)
            for r in range(8):
                for kk in range(NFEAT // 16):
                    obuf[r, pl.ds(kk * 16, 16)] = (
                        acc[pl.ds(cbase + r * NFEAT + kk * 16, 16)])
            ob = pl.multiple_of(w * ROWS_PT + c8 * 8, 8)
            pltpu.sync_copy(obuf, out_hbm.at[pl.ds(ob, 8)])
            return carry

        lax.fori_loop(0, ROWS_PT // 8, out_chunk, 0)

    return k(src, dst, x, zeros)


def _spectral_normalize(W):
    n, m = W.shape
    u = jnp.full((1, n), 1.0 / (float(n) ** 0.5), jnp.float32)
    v = jnp.full((1, m), 1.0 / (float(m) ** 0.5), jnp.float32)

    def it(i, uv):
        u, v = uv
        v = jnp.dot(u, W, preferred_element_type=jnp.float32)
        v = v / (jnp.sqrt(jnp.sum(v * v)) + 1e-12)
        u = lax.dot_general(v, W, (((1,), (1,)), ((), ())),
                            preferred_element_type=jnp.float32)
        u = u / (jnp.sqrt(jnp.sum(u * u)) + 1e-12)
        return (u, v)

    u, v = lax.fori_loop(0, 30, it, (u, v))
    sigma = jnp.sum(jnp.dot(u, W, preferred_element_type=jnp.float32) * v)
    return W / sigma


def _tc_body(x_ref, ap_ref, W1_ref, b1_ref, g_ref, be_ref, W2_ref, b2_ref, o_ref):
    W1n = _spectral_normalize(W1_ref[...])
    W2n = _spectral_normalize(W2_ref[...])
    h = x_ref[...] + ap_ref[0:N_NODES]
    h = lax.dot_general(h, W1n, (((1,), (1,)), ((), ())),
                        preferred_element_type=jnp.float32) + b1_ref[...]
    h = jnp.maximum(h, 0.0)
    mean = jnp.mean(h, axis=0, keepdims=True)
    var = jnp.mean(h * h, axis=0, keepdims=True) - mean * mean
    h = (h - mean) / jnp.sqrt(var + 1e-5) * g_ref[...] + be_ref[...]
    o_ref[...] = lax.dot_general(h, W2n, (((1,), (1,)), ((), ())),
                                 preferred_element_type=jnp.float32) + b2_ref[...]


def _tc_mlp(x, aggr_padded, W1, b1, gamma, beta, W2, b2):
    return pl.pallas_call(
        _tc_body,
        out_shape=jax.ShapeDtypeStruct((N_NODES, NHID), jnp.float32),
    )(x, aggr_padded, W1, b1.reshape(1, -1), gamma.reshape(1, -1),
      beta.reshape(1, -1), W2, b2.reshape(1, -1))


def kernel(x, edge_index, W1, b1, gamma, beta, W2, b2):
    src = jnp.pad(edge_index[0], (0, EPAD - N_EDGES))
    dst = jnp.pad(edge_index[1], (0, EPAD - N_EDGES))
    zeros = jnp.zeros((ACC_ROWS * NFEAT,), jnp.float32)
    xb = x.astype(jnp.bfloat16)
    xb = xb.reshape(N_NODES, NFEAT // 32, 2, 16).transpose(0, 1, 3, 2)
    xpk = lax.bitcast_convert_type(
        xb.reshape(N_NODES, NFEAT // 2, 2), jnp.int32)
    xpk = jnp.pad(xpk, ((0, XROWS - N_NODES), (0, 0)))
    aggr = _sc_segment_sum(src, dst, xpk, zeros)
    return _tc_mlp(x, aggr, W1, b1, gamma, beta, W2, b2)


# speedup vs baseline: 2.4682x; 1.0375x over previous
import functools

import jax
import jax.numpy as jnp
from jax import lax
from jax.experimental import pallas as pl
from jax.experimental.pallas import tpu as pltpu
from jax.experimental.pallas import tpu_sc as plsc

N_NODES = 10000
N_EDGES = 160000
NFEAT = 256
NHID = 256

NC = 2
NS = 16
NW = NC * NS
ROWS_PT = 320
ACC_ROWS = 328
TRASH = 320
XROWS = 10240
SRE = 8192
NSR = -(-N_EDGES // SRE)
EPAD = NSR * SRE
W16 = SRE // 16


def _prefix16(mi, row_iota):
    x = mi
    for k in (1, 2, 4, 8):
        idx = jnp.maximum(row_iota - k, 0)
        shifted = x.at[idx].get(mode="promise_in_bounds")
        x = jnp.where(row_iota >= k, x + shifted, x)
    return x


def _splat_last(x, row_iota):
    return x.at[row_iota * 0 + 15].get(mode="promise_in_bounds")


def _sc_segment_sum(src, dst, x, zeros):
    mesh = plsc.VectorSubcoreMesh(
        core_axis_name="c", subcore_axis_name="s", num_cores=NC, num_subcores=NS
    )

    @functools.partial(
        pl.kernel,
        mesh=mesh,
        compiler_params=pltpu.CompilerParams(needs_layout_passes=False),
        out_type=jax.ShapeDtypeStruct((NW * ROWS_PT, NFEAT), jnp.float32),
        scratch_types=[
            pltpu.VMEM((SRE,), jnp.int32),
            pltpu.VMEM((SRE,), jnp.int32),
            pltpu.VMEM((SRE + 80,), jnp.int32),
            pltpu.VMEM((SRE + 80,), jnp.int32),
            pltpu.VMEM((16, NFEAT // 2), jnp.int32),
            pltpu.VMEM((16, NFEAT // 2), jnp.int32),
            pltpu.VMEM((16, NFEAT // 2), jnp.int32),
            pltpu.VMEM((16, NFEAT // 2), jnp.int32),
            pltpu.VMEM((ACC_ROWS * NFEAT,), jnp.float32),
            pltpu.VMEM((8, NFEAT), jnp.float32),
            pltpu.SemaphoreType.DMA,
            pltpu.SemaphoreType.DMA,
            pltpu.SemaphoreType.DMA,
            pltpu.SemaphoreType.DMA,
            pltpu.SemaphoreType.DMA,
            pltpu.SemaphoreType.DMA,
        ],
    )
    def k(src_hbm, dst_hbm, x_hbm, zeros_hbm, out_hbm,
          esrc, edst, psrc, pdloc, gbufa, gbufb, gbufc, gbufd, acc, obuf,
          sema, semb, semc, semd, semi, semj):
        c = lax.axis_index("c")
        s = lax.axis_index("s")
        w = s * NC + c
        lo = w * ROWS_PT
        row_iota = lax.iota(jnp.int32, 16)
        zero16 = row_iota * 0
        junk = zero16 + (SRE + 72)
        trash = zero16 + TRASH

        pltpu.sync_copy(zeros_hbm, acc)

        pltpu.sync_copy(src_hbm.at[pl.ds(0, SRE)], esrc)
        pltpu.sync_copy(dst_hbm.at[pl.ds(0, SRE)], edst)

        def sr_body(sr, carry0):
            nchunks = jnp.minimum(W16, (N_EDGES - sr * SRE + 15) // 16)

            def chunk4(i4, cnt_v):
                parts = []
                for u in range(8):
                    o = pl.multiple_of(i4 * 128 + u * 16, 16)
                    s16 = esrc[pl.ds(o, 16)]
                    d16 = edst[pl.ds(o, 16)]
                    m = (d16 >= lo) & (d16 < lo + ROWS_PT)
                    pref = _prefix16(m.astype(jnp.int32), row_iota)
                    parts.append((s16, d16, m, pref))
                base = cnt_v
                for s16, d16, m, pref in parts:
                    pos = jnp.where(m, base + pref - 1, junk)
                    plsc.store_scatter(psrc, [pos], s16)
                    plsc.store_scatter(pdloc, [pos], d16 - lo)
                    base = base + _splat_last(pref, row_iota)
                return base

            with jax.named_scope("sc_scan"):
                cnt_v = lax.fori_loop(0, nchunks // 8, chunk4, zero16)
            plsc.store_scatter(psrc, [cnt_v + row_iota], zero16)
            plsc.store_scatter(pdloc, [cnt_v + row_iota], trash)
            t16 = (cnt_v[0] + 15) // 16

            e1 = pl.multiple_of((sr + 1) * SRE, SRE)

            @pl.when(sr + 1 < NSR)
            def _prefetch():
                pltpu.async_copy(src_hbm.at[pl.ds(e1, SRE)], esrc, semi)
                pltpu.async_copy(dst_hbm.at[pl.ds(e1, SRE)], edst, semj)

            def _start(q, gb, sem_):
                qo = pl.multiple_of(q * 16, 16)
                idxv = psrc[pl.ds(qo, 16)]
                pltpu.async_copy(x_hbm.at[idxv], gb, sem_)

            def _finish(q, gb, sem_):
                qo = pl.multiple_of(q * 16, 16)
                idxv = psrc[pl.ds(qo, 16)]
                pltpu.make_async_copy(x_hbm.at[idxv], gb, sem_).wait()
                dl16 = pdloc[pl.ds(qo, 16)]
                for j in range(16):
                    base = pl.multiple_of(dl16[j] * NFEAT, NFEAT)
                    words = [gb[j, pl.ds(kk * 16, 16)]
                             for kk in range(NFEAT // 32)]
                    for kk in range(NFEAT // 32):
                        v32 = plsc.bitcast(words[kk], jnp.bfloat16)
                        a, b = plsc.unpack(
                            v32, format=plsc.PackFormat.INTERLEAVED,
                            preferred_element_type=jnp.float32)
                        plsc.addupdate(acc.at[pl.ds(base + kk * 32, 16)], a)
                        plsc.addupdate(
                            acc.at[pl.ds(base + kk * 32 + 16, 16)], b)

            ring = ((gbufa, sema), (gbufb, semb), (gbufc, semc), (gbufd, semd))

            for b in range(3):
                @pl.when(t16 > b)
                def _prime(b=b):
                    _start(b, *ring[b])

            def flush(q, carry1):
                p = lax.rem(q, 4)
                for b in range(4):
                    @pl.when((q + 3 < t16) & (lax.rem(q + 3, 4) == b))
                    def _(b=b):
                        _start(q + 3, *ring[b])
                for b in range(4):
                    @pl.when(p == b)
                    def _(b=b):
                        _finish(q, *ring[b])
                return carry1

            with jax.named_scope("sc_flush"):
                lax.fori_loop(0, t16, flush, 0)

            @pl.when(sr + 1 < NSR)
            def _wait_prefetch():
                pltpu.make_async_copy(src_hbm.at[pl.ds(e1, SRE)], esrc,
                                      semi).wait()
                pltpu.make_async_copy(dst_hbm.at[pl.ds(e1, SRE)], edst,
                                      semj).wait()

            return carry0

        lax.fori_loop(0, NSR, sr_body, 0)
        def out_chunk(c8, carry):
            cbase = pl.multiple_of(c8 * (8 * NFEAT), 8 * NFEAT)
            for r in range(8):
                for kk in range(NFEAT // 16):
                    obuf[r, pl.ds(kk * 16, 16)] = (
                        acc[pl.ds(cbase + r * NFEAT + kk * 16, 16)])
            ob = pl.multiple_of(w * ROWS_PT + c8 * 8, 8)
            pltpu.sync_copy(obuf, out_hbm.at[pl.ds(ob, 8)])
            return carry

        lax.fori_loop(0, ROWS_PT // 8, out_chunk, 0)

    return k(src, dst, x, zeros)


def _spectral_normalize(W):
    n, m = W.shape
    u = jnp.full((1, n), 1.0 / (float(n) ** 0.5), jnp.float32)
    v = jnp.full((1, m), 1.0 / (float(m) ** 0.5), jnp.float32)

    def it(i, uv):
        u, v = uv
        v = jnp.dot(u, W, preferred_element_type=jnp.float32)
        v = v / (jnp.sqrt(jnp.sum(v * v)) + 1e-12)
        u = lax.dot_general(v, W, (((1,), (1,)), ((), ())),
                            preferred_element_type=jnp.float32)
        u = u / (jnp.sqrt(jnp.sum(u * u)) + 1e-12)
        return (u, v)

    u, v = lax.fori_loop(0, 30, it, (u, v))
    sigma = jnp.sum(jnp.dot(u, W, preferred_element_type=jnp.float32) * v)
    return W / sigma


def _tc_body(x_ref, ap_ref, W1_ref, b1_ref, g_ref, be_ref, W2_ref, b2_ref, o_ref):
    W1n = _spectral_normalize(W1_ref[...])
    W2n = _spectral_normalize(W2_ref[...])
    h = x_ref[...] + ap_ref[0:N_NODES]
    h = lax.dot_general(h, W1n, (((1,), (1,)), ((), ())),
                        preferred_element_type=jnp.float32) + b1_ref[...]
    h = jnp.maximum(h, 0.0)
    mean = jnp.mean(h, axis=0, keepdims=True)
    var = jnp.mean(h * h, axis=0, keepdims=True) - mean * mean
    h = (h - mean) / jnp.sqrt(var + 1e-5) * g_ref[...] + be_ref[...]
    o_ref[...] = lax.dot_general(h, W2n, (((1,), (1,)), ((), ())),
                                 preferred_element_type=jnp.float32) + b2_ref[...]


def _tc_mlp(x, aggr_padded, W1, b1, gamma, beta, W2, b2):
    return pl.pallas_call(
        _tc_body,
        out_shape=jax.ShapeDtypeStruct((N_NODES, NHID), jnp.float32),
    )(x, aggr_padded, W1, b1.reshape(1, -1), gamma.reshape(1, -1),
      beta.reshape(1, -1), W2, b2.reshape(1, -1))


def kernel(x, edge_index, W1, b1, gamma, beta, W2, b2):
    src = jnp.pad(edge_index[0], (0, EPAD - N_EDGES))
    dst = jnp.pad(edge_index[1], (0, EPAD - N_EDGES))
    zeros = jnp.zeros((ACC_ROWS * NFEAT,), jnp.float32)
    xb = x.astype(jnp.bfloat16)
    xb = xb.reshape(N_NODES, NFEAT // 32, 2, 16).transpose(0, 1, 3, 2)
    xpk = lax.bitcast_convert_type(
        xb.reshape(N_NODES, NFEAT // 2, 2), jnp.int32)
    xpk = jnp.pad(xpk, ((0, XROWS - N_NODES), (0, 0)))
    aggr = _sc_segment_sum(src, dst, xpk, zeros)
    return _tc_mlp(x, aggr, W1, b1, gamma, beta, W2, b2)
